# baseline (speedup 1.0000x reference)
import functools

import jax
import jax.numpy as jnp
from jax import lax
from jax.experimental import pallas as pl
from jax.experimental.pallas import tpu as pltpu
from jax.experimental.pallas import tpu_sc as plsc

_B = 16384
_NUMERIC_DIM = 16
_EMBED_DIM = 32
_OUTPUT_DIM = 128
_GRP = _OUTPUT_DIM // _EMBED_DIM

_NC = 2
_NS = 16
_NW = _NC * _NS
_BPW = _B // _NW
_CHUNK = 256
_NCHUNK = _BPW // _CHUNK


@functools.cache
def _make_sc_gather3(vg_cat, vg_host, vg_dom):
    mesh = plsc.VectorSubcoreMesh(core_axis_name="c", subcore_axis_name="s")

    @functools.partial(
        pl.kernel,
        out_type=(
            jax.ShapeDtypeStruct((_B, _OUTPUT_DIM), jnp.float32),
            jax.ShapeDtypeStruct((_B, _OUTPUT_DIM), jnp.float32),
            jax.ShapeDtypeStruct((_B, _OUTPUT_DIM), jnp.float32),
        ),
        mesh=mesh,
        scratch_types=[
            pltpu.VMEM((_CHUNK,), jnp.int32),
            pltpu.VMEM((_CHUNK,), jnp.int32),
            pltpu.VMEM((_CHUNK,), jnp.int32),
            pltpu.VMEM((_CHUNK, _OUTPUT_DIM), jnp.float32),
            pltpu.VMEM((_CHUNK, _OUTPUT_DIM), jnp.float32),
            pltpu.VMEM((_CHUNK, _OUTPUT_DIM), jnp.float32),
            pltpu.SemaphoreType.DMA,
            pltpu.SemaphoreType.DMA,
            pltpu.SemaphoreType.DMA,
        ],
    )
    def _sc_gather3(cat_idx_hbm, host_idx_hbm, dom_idx_hbm,
                    cat_tab_hbm, host_tab_hbm, dom_tab_hbm,
                    cat_out, host_out, dom_out,
                    ci_v, hi_v, di_v, cr_v, hr_v, dr_v, s0, s1, s2):
        wid = lax.axis_index("s") * _NC + lax.axis_index("c")
        base = wid * _BPW

        def step(k, _):
            off = base + k * _CHUNK
            pltpu.sync_copy(cat_idx_hbm.at[pl.ds(off, _CHUNK)], ci_v)
            pltpu.sync_copy(host_idx_hbm.at[pl.ds(off, _CHUNK)], hi_v)
            pltpu.sync_copy(dom_idx_hbm.at[pl.ds(off, _CHUNK)], di_v)
            c0 = pltpu.async_copy(cat_tab_hbm.at[ci_v], cr_v, s0)
            c1 = pltpu.async_copy(host_tab_hbm.at[hi_v], hr_v, s1)
            c2 = pltpu.async_copy(dom_tab_hbm.at[di_v], dr_v, s2)
            c0.wait()
            c1.wait()
            c2.wait()
            pltpu.sync_copy(cr_v, cat_out.at[pl.ds(off, _CHUNK)])
            pltpu.sync_copy(hr_v, host_out.at[pl.ds(off, _CHUNK)])
            pltpu.sync_copy(dr_v, dom_out.at[pl.ds(off, _CHUNK)])
            return ()

        lax.fori_loop(0, _NCHUNK, step, ())

    return _sc_gather3


_ROWS = 2048


def _select32(grp, rem):
    out = jnp.zeros((grp.shape[0], _EMBED_DIM), jnp.float32)
    for r in range(_GRP):
        sub = grp[:, r * _EMBED_DIM:(r + 1) * _EMBED_DIM]
        out = jnp.where(rem == r, sub, out)
    return out


def _tc_body(num_ref, cr_ref, hr_ref, dr_ref, cg_ref, hg_ref, dg_ref,
             w_ref, b_ref, g_ref, be_ref, out_ref):
    cat32 = _select32(cg_ref[...], cr_ref[...])
    host32 = _select32(hg_ref[...], hr_ref[...])
    dom32 = _select32(dg_ref[...], dr_ref[...])
    x = jnp.concatenate([num_ref[...], cat32, host32, dom32], axis=-1)
    h = jnp.dot(x, w_ref[...], preferred_element_type=jnp.float32) + b_ref[...]
    mean = jnp.mean(h, axis=-1, keepdims=True)
    var = jnp.mean(jnp.square(h - mean), axis=-1, keepdims=True)
    y = (h - mean) * lax.rsqrt(var + 1e-5) * g_ref[...] + be_ref[...]
    out_ref[...] = y * 0.5 * (1.0 + lax.erf(y * 0.7071067811865476))


def _tc_dense(meta_numeric, rem_cat, rem_host, rem_dom,
              cat_grp, host_grp, dom_grp, W, b, gamma, beta):
    grid = _B // _ROWS
    return pl.pallas_call(
        _tc_body,
        grid=(grid,),
        in_specs=[
            pl.BlockSpec((_ROWS, _NUMERIC_DIM), lambda i: (i, 0)),
            pl.BlockSpec((_ROWS, 1), lambda i: (i, 0)),
            pl.BlockSpec((_ROWS, 1), lambda i: (i, 0)),
            pl.BlockSpec((_ROWS, 1), lambda i: (i, 0)),
            pl.BlockSpec((_ROWS, _OUTPUT_DIM), lambda i: (i, 0)),
            pl.BlockSpec((_ROWS, _OUTPUT_DIM), lambda i: (i, 0)),
            pl.BlockSpec((_ROWS, _OUTPUT_DIM), lambda i: (i, 0)),
            pl.BlockSpec((_NUMERIC_DIM + 3 * _EMBED_DIM, _OUTPUT_DIM),
                         lambda i: (0, 0)),
            pl.BlockSpec((1, _OUTPUT_DIM), lambda i: (0, 0)),
            pl.BlockSpec((1, _OUTPUT_DIM), lambda i: (0, 0)),
            pl.BlockSpec((1, _OUTPUT_DIM), lambda i: (0, 0)),
        ],
        out_specs=pl.BlockSpec((_ROWS, _OUTPUT_DIM), lambda i: (i, 0)),
        out_shape=jax.ShapeDtypeStruct((_B, _OUTPUT_DIM), jnp.float32),
    )(meta_numeric, rem_cat, rem_host, rem_dom, cat_grp, host_grp, dom_grp,
      W, b.reshape(1, _OUTPUT_DIM), gamma.reshape(1, _OUTPUT_DIM),
      beta.reshape(1, _OUTPUT_DIM))


def kernel(meta_numeric, meta_category_id, meta_host_id, meta_domain_id,
           cat_table, host_table, domain_table, W, b, gamma, beta):
    ci = meta_category_id.astype(jnp.int32)
    hi = meta_host_id.astype(jnp.int32)
    di = meta_domain_id.astype(jnp.int32)
    cat_g = cat_table.reshape(-1, _OUTPUT_DIM)
    host_g = host_table.reshape(-1, _OUTPUT_DIM)
    dom_g = domain_table.reshape(-1, _OUTPUT_DIM)
    cat_grp, host_grp, dom_grp = _make_sc_gather3(
        cat_g.shape[0], host_g.shape[0], dom_g.shape[0])(
        ci >> 2, hi >> 2, di >> 2, cat_g, host_g, dom_g)
    return _tc_dense(meta_numeric,
                     (ci & 3).reshape(-1, 1), (hi & 3).reshape(-1, 1),
                     (di & 3).reshape(-1, 1),
                     cat_grp, host_grp, dom_grp, W, b, gamma, beta)
